# baseline (speedup 1.0000x reference)
import jax
import jax.numpy as jnp
from jax import lax
from jax.experimental import pallas as pl

N = 10000
D = 128
E = 320000


def _mm_kernel(a_ref, b_ref, bias_ref, o_ref):
    o_ref[...] = (jnp.dot(a_ref[...], b_ref[...],
                          preferred_element_type=jnp.float32)
                  + bias_ref[...])


def _mm(a, b, bias, bm):
    m, k = a.shape
    n = b.shape[1]
    return pl.pallas_call(
        _mm_kernel,
        grid=(m // bm,),
        in_specs=[
            pl.BlockSpec((bm, k), lambda i: (i, 0)),
            pl.BlockSpec((k, n), lambda i: (0, 0)),
            pl.BlockSpec((1, n), lambda i: (0, 0)),
        ],
        out_specs=pl.BlockSpec((bm, n), lambda i: (i, 0)),
        out_shape=jax.ShapeDtypeStruct((m, n), jnp.float32),
    )(a, b, bias.reshape(1, n))


def _final_kernel(dinv_ref, s_ref, h2_ref, wg_ref, bg_ref, wfc_ref, bfc_ref,
                  o_ref):
    i = pl.program_id(0)
    nblk = pl.num_programs(0)

    @pl.when(i == 0)
    def _():
        o_ref[...] = jnp.zeros_like(o_ref)

    dv = dinv_ref[...]
    sv = s_ref[...]
    w = dv * (sv + dv)
    o_ref[...] += jnp.sum(w * h2_ref[...], axis=0, keepdims=True)

    @pl.when(i == nblk - 1)
    def _():
        pooled = o_ref[...] / N
        pooled = jnp.dot(pooled, wg_ref[...],
                         preferred_element_type=jnp.float32) + bg_ref[...]
        o_ref[...] = jnp.dot(pooled, wfc_ref[...],
                             preferred_element_type=jnp.float32) + bfc_ref[...]


def _final(dinv, s, h2, Wg, bg, Wfc, bfc):
    bm = 1000
    return pl.pallas_call(
        _final_kernel,
        grid=(N // bm,),
        in_specs=[
            pl.BlockSpec((bm, 1), lambda i: (i, 0)),
            pl.BlockSpec((bm, 1), lambda i: (i, 0)),
            pl.BlockSpec((bm, D), lambda i: (i, 0)),
            pl.BlockSpec((D, D), lambda i: (0, 0)),
            pl.BlockSpec((1, D), lambda i: (0, 0)),
            pl.BlockSpec((D, D), lambda i: (0, 0)),
            pl.BlockSpec((1, D), lambda i: (0, 0)),
        ],
        out_specs=pl.BlockSpec((1, D), lambda i: (0, 0)),
        out_shape=jax.ShapeDtypeStruct((1, D), jnp.float32),
    )(dinv[:, None], s[:, None], h2, Wg, bg[None, :], Wfc, bfc[None, :])


def kernel(x, edge_index, edge_attr, W1, b1, W2, b2, Wg, bg, Wfc, bfc):
    src = edge_index[0]
    dst = edge_index[1]
    zbias = jnp.zeros((D,), jnp.float32)

    def conv(h, W, b):
        wa, wb, wc = W[:D], W[D:2 * D], W[2 * D:]
        A = _mm(h, wa - wb, zbias, 1000)
        B = _mm(h, wb, zbias, 1000)
        C = _mm(edge_attr, wc, b, 2000)
        m = jax.nn.relu(A[dst] + B[src] + C)
        return jnp.zeros((N, D), jnp.float32).at[dst].max(m)

    h1 = conv(x, W1, b1)
    h2 = conv(h1, W2, b2)

    deg = jnp.ones((N,), jnp.float32).at[dst].add(1.0)
    dinv = lax.rsqrt(deg)
    s = jnp.zeros((N,), jnp.float32).at[src].add(dinv[dst])

    return _final(dinv, s, h2, Wg, bg, Wfc, bfc)
